# Initial kernel scaffold; baseline (speedup 1.0000x reference)
import functools

import jax
import jax.numpy as jnp
from jax.experimental import pallas as pl

_TOPK = 64
_INT_MIN = jnp.int32(-2147483648)


def _block_kernel(q_ref, k_ref, v_ref, o_ref, *, top_k):
    q = q_ref[...]
    k = k_ref[...]
    d = q.shape[1]
    s = jax.lax.dot_general(
        q, k, (((1,), (1,)), ((), ())), preferred_element_type=jnp.float32
    ) * (1.0 / (d ** 0.5))

    si = jax.lax.bitcast_convert_type(s, jnp.int32)
    v_key = jnp.where(si >= 0, si, jnp.bitwise_xor(jnp.bitwise_not(si), _INT_MIN))

    def body(i, t):
        bit = jnp.int32(31) - i
        cand = jnp.bitwise_or(t, jax.lax.shift_left(jnp.int32(1), bit))
        thr = jnp.bitwise_xor(cand, _INT_MIN)
        cnt = jnp.sum((v_key >= thr).astype(jnp.int32), axis=1, keepdims=True)
        return jnp.where(cnt >= top_k, cand, t)

    t = jax.lax.fori_loop(0, 32, body, jnp.zeros((q.shape[0], 1), jnp.int32))
    mask = v_key >= jnp.bitwise_xor(t, _INT_MIN)

    w = jnp.exp(jnp.where(mask, s, 0.0)) - 1.0
    vv = v_ref[...]
    denom = jnp.float32(s.shape[1]) + jnp.sum(w, axis=1, keepdims=True)
    num = jax.lax.dot_general(
        w, vv, (((1,), (0,)), ((), ())), preferred_element_type=jnp.float32
    ) + jnp.sum(vv, axis=0, keepdims=True)
    o_ref[...] = num / denom


def kernel(Q, K, V):
    B, S, D = Q.shape
    q2 = Q.reshape(S, D)
    k2 = K.reshape(S, D)
    v2 = V.reshape(S, D)
    BM = 512
    while S % BM:
        BM //= 2
    out = pl.pallas_call(
        functools.partial(_block_kernel, top_k=_TOPK),
        grid=(S // BM,),
        in_specs=[
            pl.BlockSpec((BM, D), lambda i: (i, 0)),
            pl.BlockSpec((S, D), lambda i: (0, 0)),
            pl.BlockSpec((S, D), lambda i: (0, 0)),
        ],
        out_specs=pl.BlockSpec((BM, D), lambda i: (i, 0)),
        out_shape=jax.ShapeDtypeStruct((S, D), jnp.float32),
    )(q2, k2, v2)
    return out.reshape(B, S, D)


# speedup vs baseline: 14.7528x; 14.7528x over previous
import functools

import jax
import jax.numpy as jnp
from jax.experimental import pallas as pl

_TOPK = 64


def _block_kernel(q_ref, k_ref, v_ref, o_ref, *, top_k):
    q = q_ref[...]
    k = k_ref[...]
    d = q.shape[1]
    s = jax.lax.dot_general(
        q, k, (((1,), (1,)), ((), ())), preferred_element_type=jnp.float32
    ) * (1.0 / (d ** 0.5))

    int_min = jnp.int32(-2147483648)
    si = jax.lax.bitcast_convert_type(s, jnp.int32)
    v_key = jnp.where(si >= 0, si, jnp.bitwise_xor(jnp.bitwise_not(si), int_min))

    def body(i, t):
        bit = jnp.int32(31) - i
        cand = jnp.bitwise_or(t, jax.lax.shift_left(jnp.int32(1), bit))
        thr = jnp.bitwise_xor(cand, int_min)
        cnt = jnp.sum((v_key >= thr).astype(jnp.int32), axis=1, keepdims=True)
        return jnp.where(cnt >= top_k, cand, t)

    t = jax.lax.fori_loop(0, 32, body, jnp.zeros((q.shape[0], 1), jnp.int32))
    mask = v_key >= jnp.bitwise_xor(t, int_min)

    w = jnp.exp(jnp.where(mask, s, 0.0)) - 1.0
    vv = v_ref[...]
    denom = jnp.float32(s.shape[1]) + jnp.sum(w, axis=1, keepdims=True)
    num = jax.lax.dot_general(
        w, vv, (((1,), (0,)), ((), ())), preferred_element_type=jnp.float32
    ) + jnp.sum(vv, axis=0, keepdims=True)
    o_ref[...] = num / denom


def kernel(Q, K, V):
    B, S, D = Q.shape
    q2 = Q.reshape(S, D)
    k2 = K.reshape(S, D)
    v2 = V.reshape(S, D)
    BM = 512
    while S % BM:
        BM //= 2
    out = pl.pallas_call(
        functools.partial(_block_kernel, top_k=_TOPK),
        grid=(S // BM,),
        in_specs=[
            pl.BlockSpec((BM, D), lambda i: (i, 0)),
            pl.BlockSpec((S, D), lambda i: (0, 0)),
            pl.BlockSpec((S, D), lambda i: (0, 0)),
        ],
        out_specs=pl.BlockSpec((BM, D), lambda i: (i, 0)),
        out_shape=jax.ShapeDtypeStruct((S, D), jnp.float32),
    )(q2, k2, v2)
    return out.reshape(B, S, D)


# speedup vs baseline: 14.7793x; 1.0018x over previous
import functools

import jax
import jax.numpy as jnp
from jax.experimental import pallas as pl

_TOPK = 64


def _block_kernel(q_ref, k_ref, v_ref, o_ref, *, top_k):
    q = q_ref[...]
    k = k_ref[...]
    d = q.shape[1]
    s = jax.lax.dot_general(
        q, k, (((1,), (1,)), ((), ())), preferred_element_type=jnp.float32
    ) * (1.0 / (d ** 0.5))

    int_min = jnp.int32(-2147483648)
    si = jax.lax.bitcast_convert_type(s, jnp.int32)
    v_key = jnp.where(si >= 0, si, jnp.bitwise_xor(jnp.bitwise_not(si), int_min))

    def body(i, t):
        bit = jnp.int32(31) - i
        cand = jnp.bitwise_or(t, jax.lax.shift_left(jnp.int32(1), bit))
        thr = jnp.bitwise_xor(cand, int_min)
        cnt = jnp.sum((v_key >= thr).astype(jnp.int32), axis=1, keepdims=True)
        return jnp.where(cnt >= top_k, cand, t)

    t = jax.lax.fori_loop(0, 32, body, jnp.zeros((q.shape[0], 1), jnp.int32))
    mask = v_key >= jnp.bitwise_xor(t, int_min)

    w = jnp.exp(jnp.where(mask, s, 0.0)) - 1.0
    vv = v_ref[...]
    denom = jnp.float32(s.shape[1]) + jnp.sum(w, axis=1, keepdims=True)
    num = jax.lax.dot_general(
        w.astype(jnp.bfloat16),
        vv.astype(jnp.bfloat16),
        (((1,), (0,)), ((), ())),
        preferred_element_type=jnp.float32,
    ) + jnp.sum(vv, axis=0, keepdims=True)
    o_ref[...] = num / denom


def kernel(Q, K, V):
    B, S, D = Q.shape
    q2 = Q.reshape(S, D)
    k2 = K.reshape(S, D)
    v2 = V.reshape(S, D)
    BM = 512
    while S % BM:
        BM //= 2
    out = pl.pallas_call(
        functools.partial(_block_kernel, top_k=_TOPK),
        grid=(S // BM,),
        in_specs=[
            pl.BlockSpec((BM, D), lambda i: (i, 0)),
            pl.BlockSpec((S, D), lambda i: (0, 0)),
            pl.BlockSpec((S, D), lambda i: (0, 0)),
        ],
        out_specs=pl.BlockSpec((BM, D), lambda i: (i, 0)),
        out_shape=jax.ShapeDtypeStruct((S, D), jnp.float32),
    )(q2, k2, v2)
    return out.reshape(B, S, D)


# speedup vs baseline: 16.5887x; 1.1224x over previous
import functools

import jax
import jax.numpy as jnp
from jax.experimental import pallas as pl

_TOPK = 64


def _count_ge16(x16, thr16):
    m = (x16 >= thr16).astype(jnp.int16)
    n = x16.shape[1]
    while n > 128:
        half = n // 2
        m = m[:, :half] + m[:, half:]
        n = half
    return jnp.sum(m.astype(jnp.int32), axis=1, keepdims=True)


def _block_kernel(q_ref, k_ref, v_ref, o_ref, *, top_k):
    q = q_ref[...]
    k = k_ref[...]
    d = q.shape[1]
    s = jax.lax.dot_general(
        q, k, (((1,), (1,)), ((), ())), preferred_element_type=jnp.float32
    ) * (1.0 / (d ** 0.5))

    int_min = jnp.int32(-2147483648)
    si = jax.lax.bitcast_convert_type(s, jnp.int32)
    v_key = jnp.where(si >= 0, si, jnp.bitwise_xor(jnp.bitwise_not(si), int_min))

    rows = q.shape[0]

    hi16 = jax.lax.shift_right_arithmetic(v_key, 16).astype(jnp.int16)

    def body1(i, t_u):
        cand = jnp.bitwise_or(t_u, jax.lax.shift_left(jnp.int32(1), 15 - i))
        thr = (cand - 32768).astype(jnp.int16)
        cnt = _count_ge16(hi16, thr)
        return jnp.where(cnt >= top_k, cand, t_u)

    t_hi = jax.lax.fori_loop(0, 16, body1, jnp.zeros((rows, 1), jnp.int32))

    thr_p1 = (t_hi + 1 - 32768).astype(jnp.int16)
    c_hi_raw = _count_ge16(hi16, thr_p1)
    c_hi = jnp.where(t_hi >= 65535, 0, c_hi_raw)
    k2 = top_k - c_hi

    lo16 = jax.lax.bitwise_xor(
        jnp.bitwise_and(v_key, 65535), 32768).astype(jnp.int16)
    in_bucket = hi16 == (t_hi - 32768).astype(jnp.int16)
    ml = jnp.where(in_bucket, lo16, jnp.int16(-32768))

    def body2(i, t_u):
        cand = jnp.bitwise_or(t_u, jax.lax.shift_left(jnp.int32(1), 15 - i))
        thr = (cand - 32768).astype(jnp.int16)
        cnt = _count_ge16(ml, thr)
        return jnp.where(cnt >= k2, cand, t_u)

    t_lo = jax.lax.fori_loop(0, 16, body2, jnp.zeros((rows, 1), jnp.int32))

    thr32 = jnp.bitwise_xor(
        jnp.bitwise_or(jax.lax.shift_left(t_hi, 16), t_lo), int_min)
    mask = v_key >= thr32

    w = jnp.exp(jnp.where(mask, s, 0.0)) - 1.0
    vv = v_ref[...]
    denom = jnp.float32(s.shape[1]) + jnp.sum(w, axis=1, keepdims=True)
    num = jax.lax.dot_general(
        w.astype(jnp.bfloat16),
        vv.astype(jnp.bfloat16),
        (((1,), (0,)), ((), ())),
        preferred_element_type=jnp.float32,
    ) + jnp.sum(vv, axis=0, keepdims=True)
    o_ref[...] = num / denom


def kernel(Q, K, V):
    B, S, D = Q.shape
    q2 = Q.reshape(S, D)
    k2 = K.reshape(S, D)
    v2 = V.reshape(S, D)
    BM = 256
    while S % BM:
        BM //= 2
    out = pl.pallas_call(
        functools.partial(_block_kernel, top_k=_TOPK),
        grid=(S // BM,),
        in_specs=[
            pl.BlockSpec((BM, D), lambda i: (i, 0)),
            pl.BlockSpec((S, D), lambda i: (0, 0)),
            pl.BlockSpec((S, D), lambda i: (0, 0)),
        ],
        out_specs=pl.BlockSpec((BM, D), lambda i: (i, 0)),
        out_shape=jax.ShapeDtypeStruct((S, D), jnp.float32),
    )(q2, k2, v2)
    return out.reshape(B, S, D)
